# baseline (speedup 1.0000x reference)
import functools

import jax
import jax.numpy as jnp
from jax import lax
from jax.experimental import pallas as pl
from jax.experimental.pallas import tpu as pltpu
from jax.experimental.pallas import tpu_sc as plsc

VOCAB = 1000000
EMBED_DIM = 32
B1 = 4096
B2 = 200
B_TOTAL = B1 * B2

_info = plsc.get_sparse_core_info()
_NC, _NS = _info.num_cores, _info.num_subcores
_NW = _NC * _NS
_NBLK = (B1 // 128) * B2
_BLK_PER_W = _NBLK // _NW
_IDX_PER_W = _BLK_PER_W * 128


def _make_gather():
    mesh = plsc.VectorSubcoreMesh(core_axis_name="c", subcore_axis_name="s")

    @functools.partial(
        pl.kernel,
        mesh=mesh,
        out_type=jax.ShapeDtypeStruct((B2, 4, B1 // 128, 8, 128), jnp.float32),
        scratch_types=[
            pltpu.VMEM((_IDX_PER_W,), jnp.int32),
            pltpu.VMEM((128, EMBED_DIM), jnp.float32),
            pltpu.VMEM((128, EMBED_DIM), jnp.float32),
            pltpu.VMEM((EMBED_DIM, 128), jnp.float32),
            pltpu.VMEM((EMBED_DIM, 128), jnp.float32),
            [pltpu.SemaphoreType.DMA] * 5,
        ],
        compiler_params=pltpu.CompilerParams(
            use_tc_tiling_on_sc=False, needs_layout_passes=False
        ),
    )
    def gather_kernel(
        table_hbm, idx_hbm, out_hbm, idx_v, rows_a, rows_b, obuf_a, obuf_b, sems
    ):
        rows_ref = (rows_a, rows_b)
        obuf_ref = (obuf_a, obuf_b)
        wid = lax.axis_index("s") * _NC + lax.axis_index("c")
        si, ga, gb, oa, ob = sems
        riota = lax.iota(jnp.int32, 16)

        pltpu.make_async_copy(
            idx_hbm.at[pl.ds(wid * _IDX_PER_W, _IDX_PER_W)], idx_v, si
        ).start()
        pltpu.make_async_copy(
            idx_hbm.at[pl.ds(0, _IDX_PER_W)], idx_v, si
        ).wait()

        def start_gather(j, b, sem):
            pltpu.make_async_copy(
                table_hbm.at[idx_v.at[pl.ds(j * 128, 128)]], rows_ref[b], sem
            ).start()

        def wait_gather(sem):
            pltpu.make_async_copy(
                table_hbm.at[idx_v.at[pl.ds(0, 128)]], rows_ref[0], sem
            ).wait()

        def transpose(b):
            for e in range(EMBED_DIM):
                cvec = jnp.full((16,), e, jnp.int32)
                for b1c in range(8):
                    v = plsc.load_gather(rows_ref[b], [riota + (16 * b1c), cvec])
                    obuf_ref[b][e, pl.ds(16 * b1c, 16)] = v

        def start_out(j, b, sem):
            g = wid * _BLK_PER_W + j
            b2 = g // 32
            b1g = g % 32
            for eg in range(4):
                pltpu.make_async_copy(
                    obuf_ref[b].at[pl.ds(8 * eg, 8)], out_hbm.at[b2, eg, b1g], sem
                ).start()

        def wait_out(b, sem):
            for _eg in range(4):
                pltpu.make_async_copy(
                    obuf_ref[b].at[pl.ds(0, 8)], out_hbm.at[0, 0, 0], sem
                ).wait()

        start_gather(0, 0, ga)

        def blk_body(i, _):
            ja = 2 * i
            jb = 2 * i + 1
            start_gather(jb, 1, gb)
            wait_gather(ga)

            @pl.when(i > 0)
            def _():
                wait_out(0, oa)

            transpose(0)
            start_out(ja, 0, oa)

            @pl.when(i < _BLK_PER_W // 2 - 1)
            def _():
                start_gather(jb + 1, 0, ga)

            wait_gather(gb)

            @pl.when(i > 0)
            def _():
                wait_out(1, ob)

            transpose(1)
            start_out(jb, 1, ob)
            return _

        lax.fori_loop(0, _BLK_PER_W // 2, blk_body, None)
        wait_out(0, oa)
        wait_out(1, ob)

    return gather_kernel


_gather = _make_gather()


@jax.jit
def kernel(code_tokens, embedding_table):
    idx = code_tokens.T.reshape(B_TOTAL).astype(jnp.int32)
    tab = jax.lax.optimization_barrier(
        embedding_table.reshape(VOCAB // 4, 4 * EMBED_DIM)
    ).reshape(VOCAB, EMBED_DIM)
    out5 = _gather(tab, idx)
    return out5.transpose(2, 4, 0, 1, 3).reshape(B1, B2, EMBED_DIM)


# speedup vs baseline: 1.7188x; 1.7188x over previous
import functools

import jax
import jax.numpy as jnp
from jax import lax
from jax.experimental import pallas as pl
from jax.experimental.pallas import tpu as pltpu
from jax.experimental.pallas import tpu_sc as plsc

VOCAB = 1000000
EMBED_DIM = 32
B1 = 4096
B2 = 200
B_TOTAL = B1 * B2

_info = plsc.get_sparse_core_info()
_NC, _NS = _info.num_cores, _info.num_subcores
_NW = _NC * _NS
_NBLK = (B1 // 128) * B2
_BLK_PER_W = _NBLK // _NW
_IDX_PER_W = _BLK_PER_W * 128


def _make_gather():
    mesh = plsc.VectorSubcoreMesh(core_axis_name="c", subcore_axis_name="s")

    @functools.partial(
        pl.kernel,
        mesh=mesh,
        out_type=jax.ShapeDtypeStruct((B2, 4, B1 // 128, 8, 128), jnp.float32),
        scratch_types=[
            pltpu.VMEM((_IDX_PER_W,), jnp.int32),
            pltpu.VMEM((128, EMBED_DIM), jnp.float32),
            pltpu.VMEM((128, EMBED_DIM), jnp.float32),
            pltpu.VMEM((EMBED_DIM, 133), jnp.float32),
            pltpu.VMEM((EMBED_DIM, 133), jnp.float32),
            [pltpu.SemaphoreType.DMA] * 5,
        ],
        compiler_params=pltpu.CompilerParams(
            use_tc_tiling_on_sc=False, needs_layout_passes=False
        ),
    )
    def gather_kernel(
        table_hbm, idx_hbm, out_hbm, idx_v, rows_a, rows_b, obuf_a, obuf_b, sems
    ):
        rows_ref = (rows_a, rows_b)
        obuf_ref = (obuf_a, obuf_b)
        wid = lax.axis_index("s") * _NC + lax.axis_index("c")
        si, ga, gb, oa, ob = sems
        riota = lax.iota(jnp.int32, 16)

        pltpu.make_async_copy(
            idx_hbm.at[pl.ds(wid * _IDX_PER_W, _IDX_PER_W)], idx_v, si
        ).start()
        pltpu.make_async_copy(
            idx_hbm.at[pl.ds(0, _IDX_PER_W)], idx_v, si
        ).wait()

        def start_gather(j, b, sem):
            pltpu.make_async_copy(
                table_hbm.at[idx_v.at[pl.ds(j * 128, 128)]], rows_ref[b], sem
            ).start()

        def wait_gather(sem):
            pltpu.make_async_copy(
                table_hbm.at[idx_v.at[pl.ds(0, 128)]], rows_ref[0], sem
            ).wait()

        def transpose(b):
            def t_body(i, _):
                for u in range(8):
                    b1r = i * 8 + u
                    bvec = jnp.zeros((16,), jnp.int32) + b1r
                    for h in range(2):
                        v = rows_ref[b][b1r, pl.ds(16 * h, 16)]
                        plsc.store_scatter(obuf_ref[b], [riota + 16 * h, bvec], v)
                return _

            lax.fori_loop(0, 16, t_body, None)

        def start_out(j, b, sem):
            g = wid * _BLK_PER_W + j
            b2 = g // 32
            b1g = g % 32
            for eg in range(4):
                pltpu.make_async_copy(
                    obuf_ref[b].at[pl.ds(8 * eg, 8), pl.ds(0, 128)],
                    out_hbm.at[b2, eg, b1g],
                    sem,
                ).start()

        def wait_out(b, sem):
            for _eg in range(4):
                pltpu.make_async_copy(
                    obuf_ref[b].at[pl.ds(0, 8), pl.ds(0, 128)],
                    out_hbm.at[0, 0, 0],
                    sem,
                ).wait()

        start_gather(0, 0, ga)

        def blk_body(i, _):
            ja = 2 * i
            jb = 2 * i + 1
            start_gather(jb, 1, gb)
            wait_gather(ga)

            @pl.when(i > 0)
            def _():
                wait_out(0, oa)

            transpose(0)
            start_out(ja, 0, oa)

            @pl.when(i < _BLK_PER_W // 2 - 1)
            def _():
                start_gather(jb + 1, 0, ga)

            wait_gather(gb)

            @pl.when(i > 0)
            def _():
                wait_out(1, ob)

            transpose(1)
            start_out(jb, 1, ob)
            return _

        lax.fori_loop(0, _BLK_PER_W // 2, blk_body, None)
        wait_out(0, oa)
        wait_out(1, ob)

    return gather_kernel


_gather = _make_gather()


@jax.jit
def kernel(code_tokens, embedding_table):
    idx = code_tokens.T.reshape(B_TOTAL).astype(jnp.int32)
    tab = jax.lax.optimization_barrier(
        embedding_table.reshape(VOCAB // 4, 4 * EMBED_DIM)
    ).reshape(VOCAB, EMBED_DIM)
    out5 = _gather(tab, idx)
    return out5.transpose(2, 4, 0, 1, 3).reshape(B1, B2, EMBED_DIM)
